# Initial kernel scaffold; baseline (speedup 1.0000x reference)
import jax
import jax.numpy as jnp
from jax.experimental import pallas as pl


def kernel(x, edge_index, edge_type, W1, b1, W2, b2, Wc, bc):
    raise NotImplementedError("write your pallas kernel here")


# speedup vs baseline: 56.5309x; 56.5309x over previous
import functools

import jax
import jax.numpy as jnp
from jax import lax
from jax.experimental import pallas as pl
from jax.experimental.pallas import tpu as pltpu
from jax.experimental.pallas import tpu_sc as plsc

N = 100000
R = 4
F = 8
H = 32
NP = 100352
DEGSZ = 4 * NP + 4096
OWNSZ = 2 * NP + 2048
EP = 1605632
C = 512
NB = 1024


def _mesh():
    return plsc.VectorSubcoreMesh(core_axis_name="c", subcore_axis_name="s")


def _pass_a_body(src_hbm, dst_hbm, typ_hbm, degs_out,
                 deg_sp, zbuf, sb_s, sb_d, sb_t, kb, ones_v, sem):
    cid = lax.axis_index("c")
    sid = lax.axis_index("s")
    wid = cid * 16 + sid

    @pl.loop(0, 4224, step=16)
    def _zero_zbuf(i):
        zbuf[pl.ds(i, 16)] = jnp.zeros((16,), jnp.float32)

    @pl.loop(0, 128, step=16)
    def _fill_ones(i):
        ones_v[pl.ds(i, 16)] = jnp.ones((16,), jnp.float32)

    slice_w = (2 * DEGSZ) // 16
    for j in range(12):
        pltpu.sync_copy(zbuf, deg_sp.at[pl.ds(sid * slice_w + j * 4224, 4224)])
    plsc.subcore_barrier()

    ept = EP // 32
    base = wid * ept

    @pl.loop(0, 98)
    def _chunk(c):
        off = base + c * C
        pltpu.sync_copy(src_hbm.at[pl.ds(off, C)], sb_s)
        pltpu.sync_copy(dst_hbm.at[pl.ds(off, C)], sb_d)
        pltpu.sync_copy(typ_hbm.at[pl.ds(off, C)], sb_t)
        for j in range(32):
            s16 = sb_s[pl.ds(16 * j, 16)]
            d16 = sb_d[pl.ds(16 * j, 16)]
            t16 = sb_t[pl.ds(16 * j, 16)]
            tn = t16 * NP
            row, col = j // 8, (j % 8) * 16
            kb[row, pl.ds(col, 16)] = tn + s16
            kb[4 + row, pl.ds(col, 16)] = (DEGSZ + tn) + d16
        cps = [pltpu.async_copy(ones_v, deg_sp.at[kb.at[r]], sem, add=True)
               for r in range(8)]
        for cp in cps:
            cp.wait()

    plsc.subcore_barrier()
    pltpu.sync_copy(deg_sp.at[pl.ds(sid * slice_w, slice_w)],
                    degs_out.at[cid, pl.ds(sid * slice_w, slice_w)])


def _pass_a(srcp, dstp, typp):
    k = pl.kernel(
        _pass_a_body,
        out_type=jax.ShapeDtypeStruct((2, 2 * DEGSZ), jnp.float32),
        mesh=_mesh(),
        scratch_types=[
            pltpu.VMEM_SHARED((2 * DEGSZ,), jnp.float32),
            pltpu.VMEM((4224,), jnp.float32),
            pltpu.VMEM((C,), jnp.int32),
            pltpu.VMEM((C,), jnp.int32),
            pltpu.VMEM((C,), jnp.int32),
            pltpu.VMEM((8, 128), jnp.int32),
            pltpu.VMEM((128,), jnp.float32),
            pltpu.SemaphoreType.DMA,
        ],
    )
    return k(srcp, dstp, typp)


def _pass_b_body(src_hbm, dst_hbm, typ_hbm, xs_hbm, nd_hbm,
                 agg_out, c_out,
                 agg_sp, c_sp, zb1, zb2,
                 sb_s, sb_d, sb_t, kg, ka, kc, kn, rows_v, ndv_v, sem):
    cid = lax.axis_index("c")
    sid = lax.axis_index("s")

    @pl.loop(0, 4224, step=16)
    def _zero_zb1(i):
        zb1[pl.ds(i, 16)] = jnp.zeros((16,), jnp.float32)

    @pl.loop(0, 528, step=2)
    def _zero_zb2(i):
        zb2[i, :] = jnp.zeros((8,), jnp.float32)
        zb2[i + 1, :] = jnp.zeros((8,), jnp.float32)

    rows_per_tile = OWNSZ // 16
    for j in range(24):
        pltpu.sync_copy(
            zb2, agg_sp.at[pl.ds(sid * rows_per_tile + j * 528, 528), :])
    for j in range(3):
        pltpu.sync_copy(
            zb1, c_sp.at[pl.ds(sid * rows_per_tile + j * 4224, 4224)])
    plsc.subcore_barrier()

    ept = EP // 16
    base = sid * ept

    @pl.loop(0, 196)
    def _chunk(c):
        off = base + c * C
        pltpu.sync_copy(src_hbm.at[pl.ds(off, C)], sb_s)
        pltpu.sync_copy(dst_hbm.at[pl.ds(off, C)], sb_d)
        pltpu.sync_copy(typ_hbm.at[pl.ds(off, C)], sb_t)
        for j in range(32):
            s16 = sb_s[pl.ds(16 * j, 16)]
            d16 = sb_d[pl.ds(16 * j, 16)]
            t16 = sb_t[pl.ds(16 * j, 16)]
            own = (t16 >> 1) == cid
            tn = (t16 & 1) * NP
            kgv = jnp.where(own, t16 * NP + s16, (4 * NP) + (s16 & 2047))
            kav = jnp.where(own, tn + d16, (2 * NP) + (d16 & 2047))
            kcv = jnp.where(own, tn + s16, (2 * NP) + (s16 & 2047))
            knv = jnp.where(own, t16 * NP + d16, (4 * NP) + (d16 & 2047))
            row, col = j // 8, (j % 8) * 16
            kg[row, pl.ds(col, 16)] = kgv
            ka[row, pl.ds(col, 16)] = kav
            kc[row, pl.ds(col, 16)] = kcv
            kn[row, pl.ds(col, 16)] = knv
        gcps = [pltpu.async_copy(xs_hbm.at[kg.at[r]],
                                 rows_v.at[pl.ds(r * 128, 128), :], sem)
                for r in range(4)]
        gcps += [pltpu.async_copy(nd_hbm.at[kn.at[r]],
                                  ndv_v.at[pl.ds(r * 128, 128)], sem)
                 for r in range(4)]
        for cp in gcps:
            cp.wait()
        scps = [pltpu.async_copy(rows_v.at[pl.ds(r * 128, 128), :],
                                 agg_sp.at[ka.at[r]], sem, add=True)
                for r in range(4)]
        scps += [pltpu.async_copy(ndv_v.at[pl.ds(r * 128, 128)],
                                  c_sp.at[kc.at[r]], sem, add=True)
                 for r in range(4)]
        for cp in scps:
            cp.wait()

    plsc.subcore_barrier()
    pltpu.sync_copy(agg_sp.at[pl.ds(sid * rows_per_tile, rows_per_tile), :],
                    agg_out.at[cid, pl.ds(sid * rows_per_tile, rows_per_tile), :])
    pltpu.sync_copy(c_sp.at[pl.ds(sid * rows_per_tile, rows_per_tile)],
                    c_out.at[cid, pl.ds(sid * rows_per_tile, rows_per_tile)])


def _pass_b(srcp, dstp, typp, xs, ndv):
    k = pl.kernel(
        _pass_b_body,
        out_type=(jax.ShapeDtypeStruct((2, OWNSZ, F), jnp.float32),
                  jax.ShapeDtypeStruct((2, OWNSZ), jnp.float32)),
        mesh=_mesh(),
        compiler_params=pltpu.CompilerParams(use_tc_tiling_on_sc=False),
        scratch_types=[
            pltpu.VMEM_SHARED((OWNSZ, F), jnp.float32),
            pltpu.VMEM_SHARED((OWNSZ,), jnp.float32),
            pltpu.VMEM((4224,), jnp.float32),
            pltpu.VMEM((528, F), jnp.float32),
            pltpu.VMEM((C,), jnp.int32),
            pltpu.VMEM((C,), jnp.int32),
            pltpu.VMEM((C,), jnp.int32),
            pltpu.VMEM((4, 128), jnp.int32),
            pltpu.VMEM((4, 128), jnp.int32),
            pltpu.VMEM((4, 128), jnp.int32),
            pltpu.VMEM((4, 128), jnp.int32),
            pltpu.VMEM((C, F), jnp.float32),
            pltpu.VMEM((C,), jnp.float32),
            pltpu.SemaphoreType.DMA,
        ],
    )
    return k(srcp, dstp, typp, xs, ndv)


def _tc1_body(degs_ref, dcol_ref, x_ref, ns_ref, nd_ref, xs_ref):
    b = pl.program_id(0)
    do_ = degs_ref[0, 0, :] + degs_ref[1, 0, :]
    di_ = degs_ref[0, 1, :] + degs_ref[1, 1, :]
    ns_ref[...] = jnp.where(do_ > 0, lax.rsqrt(jnp.maximum(do_, 1e-12)), 0.0)
    nd_ref[...] = jnp.where(di_ > 0, lax.rsqrt(jnp.maximum(di_, 1e-12)), 0.0)
    dc = dcol_ref[0, :, :] + dcol_ref[1, :, :]
    ns2 = jnp.where(dc > 0, lax.rsqrt(jnp.maximum(dc, 1e-12)), 0.0)
    valid = b < (4 * NP) // NB
    xs_ref[...] = jnp.where(valid, x_ref[...] * ns2, 0.0)


def _tc1(degs3, dcol, x_pad):
    nblk = DEGSZ // NB
    return pl.pallas_call(
        _tc1_body,
        grid=(nblk,),
        in_specs=[
            pl.BlockSpec((2, 2, NB), lambda b: (0, 0, b)),
            pl.BlockSpec((2, NB, 1), lambda b: (0, b, 0)),
            pl.BlockSpec((NB, F), lambda b: (b % (NP // NB), 0)),
        ],
        out_specs=[
            pl.BlockSpec((NB,), lambda b: (b,)),
            pl.BlockSpec((NB,), lambda b: (b,)),
            pl.BlockSpec((NB, F), lambda b: (b, 0)),
        ],
        out_shape=[
            jax.ShapeDtypeStruct((DEGSZ,), jnp.float32),
            jax.ShapeDtypeStruct((DEGSZ,), jnp.float32),
            jax.ShapeDtypeStruct((DEGSZ, F), jnp.float32),
        ],
    )(degs3, dcol, x_pad)


def _tc2_body(a0, a1, a2, a3, n0, n1, n2, n3, s0, s1, s2, s3,
              c0, c1, c2, c3, w1_ref, b1_ref, w2_ref, b2_ref,
              wc_ref, bc_ref, out_ref, acc_ref):
    b = pl.program_id(0)
    nblk = pl.num_programs(0)

    @pl.when(b == 0)
    def _init():
        acc_ref[...] = jnp.zeros_like(acc_ref)

    hpre = jnp.broadcast_to(jnp.sum(b1_ref[...], axis=0)[None, :], (NB, H))
    aggs = (a0, a1, a2, a3)
    nds = (n0, n1, n2, n3)
    for r in range(R):
        ar = aggs[r][0, :, :] * nds[r][...]
        hpre = hpre + lax.dot_general(
            ar, w1_ref[r], (((1,), (0,)), ((), ())),
            preferred_element_type=jnp.float32)
    h = jnp.maximum(hpre, 0.0)
    nss = (s0, s1, s2, s3)
    cs = (c0, c1, c2, c3)
    for r in range(R):
        tr = (nss[r][...] * cs[r][...])[None, :]
        m = lax.dot_general(tr, h, (((1,), (0,)), ((), ())),
                            preferred_element_type=jnp.float32,
                            precision=lax.Precision.HIGHEST)
        acc_ref[r, :] = acc_ref[r, :] + m[0, :]

    @pl.when(b == nblk - 1)
    def _fin():
        g = jnp.zeros((1, H), jnp.float32)
        for r in range(R):
            g = g + lax.dot_general(
                acc_ref[r, :][None, :], w2_ref[r], (((1,), (0,)), ((), ())),
                preferred_element_type=jnp.float32)
        g = g * (1.0 / N) + jnp.sum(b2_ref[...], axis=0)[None, :]
        out_ref[...] = (lax.dot_general(
            g, wc_ref[...], (((1,), (0,)), ((), ())),
            preferred_element_type=jnp.float32) + bc_ref[...])


def _tc2(agg_out, ndc, nsv, c_out, W1, b1, W2, b2, Wc, bc2):
    nblk = NP // NB
    agg_specs = [
        pl.BlockSpec((1, NB, F),
                     functools.partial(lambda r, b: (r // 2, (r % 2) * nblk + b, 0), r))
        for r in range(R)
    ]
    nd_specs = [
        pl.BlockSpec((NB, 1),
                     functools.partial(lambda r, b: (r * nblk + b, 0), r))
        for r in range(R)
    ]
    ns_specs = [
        pl.BlockSpec((NB,),
                     functools.partial(lambda r, b: (r * nblk + b,), r))
        for r in range(R)
    ]
    ownblk = OWNSZ // NB
    c_specs = [
        pl.BlockSpec((NB,),
                     functools.partial(
                         lambda r, b: ((r // 2) * ownblk + (r % 2) * nblk + b,), r))
        for r in range(R)
    ]
    full = lambda *s: pl.BlockSpec(s, lambda b: tuple(0 for _ in s))
    return pl.pallas_call(
        _tc2_body,
        grid=(nblk,),
        in_specs=(agg_specs + nd_specs + ns_specs + c_specs
                  + [full(R, F, H), full(R, H), full(R, H, H), full(R, H),
                     full(H, 2), full(1, 2)]),
        out_specs=pl.BlockSpec((1, 2), lambda b: (0, 0)),
        out_shape=jax.ShapeDtypeStruct((1, 2), jnp.float32),
        scratch_shapes=[pltpu.VMEM((R, H), jnp.float32)],
    )(*([agg_out] * R), *([ndc] * R), *([nsv] * R), *([c_out.reshape(-1)] * R),
      W1, b1, W2, b2, Wc, bc2)


def kernel(x, edge_index, edge_type, W1, b1, W2, b2, Wc, bc):
    src = edge_index[0].astype(jnp.int32)
    dst = edge_index[1].astype(jnp.int32)
    et = edge_type.astype(jnp.int32)
    e = src.shape[0]
    pad = EP - e
    spread = jnp.arange(pad, dtype=jnp.int32) & 2047
    srcp = jnp.concatenate([src, spread])
    dstp = jnp.concatenate([dst, spread])
    typp = jnp.concatenate([et, jnp.full((pad,), R, jnp.int32)])
    x_pad = jnp.pad(x.astype(jnp.float32), ((0, NP - N), (0, 0)))

    degs = _pass_a(srcp, dstp, typp)
    degs3 = degs.reshape(2, 2, DEGSZ)
    dcol = degs3[:, 0, :].reshape(2, DEGSZ, 1)
    nsv, ndv, xs = _tc1(degs3, dcol, x_pad)
    agg_out, c_out = _pass_b(srcp, dstp, typp, xs, ndv)
    out2 = _tc2(agg_out, ndv.reshape(DEGSZ, 1), nsv, c_out,
                W1.astype(jnp.float32), b1.astype(jnp.float32),
                W2.astype(jnp.float32), b2.astype(jnp.float32),
                Wc.astype(jnp.float32), bc.astype(jnp.float32).reshape(1, 2))
    return out2.reshape(2)


# speedup vs baseline: 67.6481x; 1.1967x over previous
import functools

import jax
import jax.numpy as jnp
from jax import lax
from jax.experimental import pallas as pl
from jax.experimental.pallas import tpu as pltpu
from jax.experimental.pallas import tpu_sc as plsc

N = 100000
R = 4
F = 8
H = 32
NP = 100352
DEGSZ = 4 * NP + 4096
OWNSZ = 2 * NP + 2048
EP = 1605632
C = 512
NB = 1024


def _mesh():
    return plsc.VectorSubcoreMesh(core_axis_name="c", subcore_axis_name="s")


def _pass_a_body(src_hbm, dst_hbm, typ_hbm, degs_out,
                 deg_sp, zbuf, sb_s, sb_d, sb_t, kb, ones_v, sem):
    cid = lax.axis_index("c")
    sid = lax.axis_index("s")
    wid = cid * 16 + sid

    @pl.loop(0, 4224, step=16)
    def _zero_zbuf(i):
        zbuf[pl.ds(i, 16)] = jnp.zeros((16,), jnp.float32)

    @pl.loop(0, 128, step=16)
    def _fill_ones(i):
        ones_v[pl.ds(i, 16)] = jnp.ones((16,), jnp.float32)

    slice_w = (2 * DEGSZ) // 16
    for j in range(12):
        pltpu.sync_copy(zbuf, deg_sp.at[pl.ds(sid * slice_w + j * 4224, 4224)])
    plsc.subcore_barrier()

    ept = EP // 32
    base = wid * ept

    @pl.loop(0, 98)
    def _chunk(c):
        off = base + c * C
        pltpu.sync_copy(src_hbm.at[pl.ds(off, C)], sb_s)
        pltpu.sync_copy(dst_hbm.at[pl.ds(off, C)], sb_d)
        pltpu.sync_copy(typ_hbm.at[pl.ds(off, C)], sb_t)
        for j in range(32):
            s16 = sb_s[pl.ds(16 * j, 16)]
            d16 = sb_d[pl.ds(16 * j, 16)]
            t16 = sb_t[pl.ds(16 * j, 16)]
            tn = t16 * NP
            row, col = j // 8, (j % 8) * 16
            kb[row, pl.ds(col, 16)] = tn + s16
            kb[4 + row, pl.ds(col, 16)] = (DEGSZ + tn) + d16
        cps = [pltpu.async_copy(ones_v, deg_sp.at[kb.at[r]], sem, add=True)
               for r in range(8)]
        for cp in cps:
            cp.wait()

    plsc.subcore_barrier()
    pltpu.sync_copy(deg_sp.at[pl.ds(sid * slice_w, slice_w)],
                    degs_out.at[cid, pl.ds(sid * slice_w, slice_w)])


def _pass_a(srcp, dstp, typp):
    k = pl.kernel(
        _pass_a_body,
        out_type=jax.ShapeDtypeStruct((2, 2 * DEGSZ), jnp.float32),
        mesh=_mesh(),
        scratch_types=[
            pltpu.VMEM_SHARED((2 * DEGSZ,), jnp.float32),
            pltpu.VMEM((4224,), jnp.float32),
            pltpu.VMEM((C,), jnp.int32),
            pltpu.VMEM((C,), jnp.int32),
            pltpu.VMEM((C,), jnp.int32),
            pltpu.VMEM((8, 128), jnp.int32),
            pltpu.VMEM((128,), jnp.float32),
            pltpu.SemaphoreType.DMA,
        ],
    )
    return k(srcp, dstp, typp)


def _pass_b_body(src_hbm, dst_hbm, typ_hbm, xs_hbm, nd_hbm,
                 agg_out, c_out,
                 agg_sp, c_sp, zb1, zb2,
                 sb_s, sb_d, sb_t, kg, ka, kc, kn, rows_v, ndv_v, sem):
    cid = lax.axis_index("c")
    sid = lax.axis_index("s")

    @pl.loop(0, 4224, step=16)
    def _zero_zb1(i):
        zb1[pl.ds(i, 16)] = jnp.zeros((16,), jnp.float32)

    @pl.loop(0, 528, step=2)
    def _zero_zb2(i):
        zb2[i, :] = jnp.zeros((8,), jnp.float32)
        zb2[i + 1, :] = jnp.zeros((8,), jnp.float32)

    rows_per_tile = OWNSZ // 16
    for j in range(24):
        pltpu.sync_copy(
            zb2, agg_sp.at[pl.ds(sid * rows_per_tile + j * 528, 528), :])
    for j in range(3):
        pltpu.sync_copy(
            zb1, c_sp.at[pl.ds(sid * rows_per_tile + j * 4224, 4224)])
    plsc.subcore_barrier()

    ept = EP // 16
    base = sid * ept

    @pl.loop(0, 196)
    def _chunk(c):
        off = base + c * C
        pltpu.sync_copy(src_hbm.at[pl.ds(off, C)], sb_s)
        pltpu.sync_copy(dst_hbm.at[pl.ds(off, C)], sb_d)
        pltpu.sync_copy(typ_hbm.at[pl.ds(off, C)], sb_t)
        for j in range(32):
            s16 = sb_s[pl.ds(16 * j, 16)]
            d16 = sb_d[pl.ds(16 * j, 16)]
            t16 = sb_t[pl.ds(16 * j, 16)]
            own = (t16 >> 1) == cid
            tn = (t16 & 1) * NP
            kgv = jnp.where(own, t16 * NP + s16, (4 * NP) + (s16 & 2047))
            kav = jnp.where(own, tn + d16, (2 * NP) + (d16 & 2047))
            kcv = jnp.where(own, tn + s16, (2 * NP) + (s16 & 2047))
            knv = jnp.where(own, t16 * NP + d16, (4 * NP) + (d16 & 2047))
            row, col = j // 8, (j % 8) * 16
            kg[row, pl.ds(col, 16)] = kgv
            ka[row, pl.ds(col, 16)] = kav
            kc[row, pl.ds(col, 16)] = kcv
            kn[row, pl.ds(col, 16)] = knv
        gcps = [pltpu.async_copy(xs_hbm.at[kg.at[r]],
                                 rows_v.at[pl.ds(r * 128, 128), :], sem)
                for r in range(4)]
        gcps += [pltpu.async_copy(nd_hbm.at[kn.at[r]],
                                  ndv_v.at[pl.ds(r * 128, 128)], sem)
                 for r in range(4)]
        for cp in gcps:
            cp.wait()
        scps = [pltpu.async_copy(rows_v.at[pl.ds(r * 128, 128), :],
                                 agg_sp.at[ka.at[r]], sem, add=True)
                for r in range(4)]
        scps += [pltpu.async_copy(ndv_v.at[pl.ds(r * 128, 128)],
                                  c_sp.at[kc.at[r]], sem, add=True)
                 for r in range(4)]
        for cp in scps:
            cp.wait()

    plsc.subcore_barrier()
    pltpu.sync_copy(agg_sp.at[pl.ds(sid * rows_per_tile, rows_per_tile), :],
                    agg_out.at[cid, pl.ds(sid * rows_per_tile, rows_per_tile), :])
    pltpu.sync_copy(c_sp.at[pl.ds(sid * rows_per_tile, rows_per_tile)],
                    c_out.at[cid, pl.ds(sid * rows_per_tile, rows_per_tile)])


def _pass_b(srcp, dstp, typp, xs, ndv):
    k = pl.kernel(
        _pass_b_body,
        out_type=(jax.ShapeDtypeStruct((2, OWNSZ, F), jnp.float32),
                  jax.ShapeDtypeStruct((2, OWNSZ), jnp.float32)),
        mesh=_mesh(),
        compiler_params=pltpu.CompilerParams(use_tc_tiling_on_sc=False),
        scratch_types=[
            pltpu.VMEM_SHARED((OWNSZ, F), jnp.float32),
            pltpu.VMEM_SHARED((OWNSZ,), jnp.float32),
            pltpu.VMEM((4224,), jnp.float32),
            pltpu.VMEM((528, F), jnp.float32),
            pltpu.VMEM((C,), jnp.int32),
            pltpu.VMEM((C,), jnp.int32),
            pltpu.VMEM((C,), jnp.int32),
            pltpu.VMEM((4, 128), jnp.int32),
            pltpu.VMEM((4, 128), jnp.int32),
            pltpu.VMEM((4, 128), jnp.int32),
            pltpu.VMEM((4, 128), jnp.int32),
            pltpu.VMEM((C, F), jnp.float32),
            pltpu.VMEM((C,), jnp.float32),
            pltpu.SemaphoreType.DMA,
        ],
    )
    return k(srcp, dstp, typp, xs, ndv)


def _rep8(v64x16):
    e = (jnp.arange(128)[None, :] // 8 == jnp.arange(16)[:, None]).astype(jnp.float32)
    return lax.dot_general(v64x16, e, (((1,), (0,)), ((), ())),
                           preferred_element_type=jnp.float32)


def _norm(d):
    return jnp.where(d > 0, lax.rsqrt(jnp.maximum(d, 1e-12)), 0.0)


def _tc1_body(degs_ref, degv_ref, x_ref, ns_ref, nd_ref, xs_ref):
    b = pl.program_id(0)
    do_ = degs_ref[0, 0, :] + degs_ref[1, 0, :]
    di_ = degs_ref[0, 1, :] + degs_ref[1, 1, :]
    ns_ref[...] = _norm(do_)
    nd_ref[...] = _norm(di_)
    d64 = degv_ref[0, 0, :, :] + degv_ref[1, 0, :, :]
    nsrep = _rep8(_norm(d64))
    valid = b < (4 * NP) // NB
    xs_ref[...] = jnp.where(valid, x_ref[...] * nsrep, 0.0)


def _tc1(degs3, degs4, x_flat2d):
    nblk = DEGSZ // NB
    return pl.pallas_call(
        _tc1_body,
        grid=(nblk,),
        in_specs=[
            pl.BlockSpec((2, 2, NB), lambda b: (0, 0, b)),
            pl.BlockSpec((2, 2, 64, 16), lambda b: (0, 0, b, 0)),
            pl.BlockSpec((64, 128), lambda b: (b % (NP // NB), 0)),
        ],
        out_specs=[
            pl.BlockSpec((NB,), lambda b: (b,)),
            pl.BlockSpec((NB,), lambda b: (b,)),
            pl.BlockSpec((64, 128), lambda b: (b, 0)),
        ],
        out_shape=[
            jax.ShapeDtypeStruct((DEGSZ,), jnp.float32),
            jax.ShapeDtypeStruct((DEGSZ,), jnp.float32),
            jax.ShapeDtypeStruct((DEGSZ * F // 128, 128), jnp.float32),
        ],
    )(degs3, degs4, x_flat2d)


def _tc15_body(agg_ref, nd_ref, out_ref):
    out_ref[...] = agg_ref[...] * _rep8(nd_ref[...])


def _tc15(agg2d, ndre2d):
    nrows = 2 * OWNSZ * F // 128
    return pl.pallas_call(
        _tc15_body,
        grid=(nrows // 64,),
        in_specs=[
            pl.BlockSpec((64, 128), lambda b: (b, 0)),
            pl.BlockSpec((64, 16), lambda b: (b, 0)),
        ],
        out_specs=pl.BlockSpec((64, 128), lambda b: (b, 0)),
        out_shape=jax.ShapeDtypeStruct((nrows, 128), jnp.float32),
    )(agg2d, ndre2d)


def _tc2_body(a0, a1, a2, a3, s0, s1, s2, s3,
              c0, c1, c2, c3, w1_ref, b1_ref, w2_ref, b2_ref,
              wc_ref, bc_ref, out_ref, acc_ref):
    b = pl.program_id(0)
    nblk = pl.num_programs(0)

    @pl.when(b == 0)
    def _init():
        acc_ref[...] = jnp.zeros_like(acc_ref)

    hpre = jnp.broadcast_to(jnp.sum(b1_ref[...], axis=0)[None, :], (NB, H))
    aggs = (a0, a1, a2, a3)
    for r in range(R):
        hpre = hpre + lax.dot_general(
            aggs[r][0, :, :], w1_ref[r], (((1,), (0,)), ((), ())),
            preferred_element_type=jnp.float32)
    h = jnp.maximum(hpre, 0.0)
    nss = (s0, s1, s2, s3)
    cs = (c0, c1, c2, c3)
    for r in range(R):
        tr = (nss[r][...] * cs[r][...])[None, :]
        m = lax.dot_general(tr, h, (((1,), (0,)), ((), ())),
                            preferred_element_type=jnp.float32,
                            precision=lax.Precision.HIGHEST)
        acc_ref[r, :] = acc_ref[r, :] + m[0, :]

    @pl.when(b == nblk - 1)
    def _fin():
        g = jnp.zeros((1, H), jnp.float32)
        for r in range(R):
            g = g + lax.dot_general(
                acc_ref[r, :][None, :], w2_ref[r], (((1,), (0,)), ((), ())),
                preferred_element_type=jnp.float32)
        g = g * (1.0 / N) + jnp.sum(b2_ref[...], axis=0)[None, :]
        out_ref[...] = (lax.dot_general(
            g, wc_ref[...], (((1,), (0,)), ((), ())),
            preferred_element_type=jnp.float32) + bc_ref[...])


def _tc2(aggn, nsv, c_flat, W1, b1, W2, b2, Wc, bc2):
    nblk = NP // NB
    agg_specs = [
        pl.BlockSpec((1, NB, F),
                     functools.partial(lambda r, b: (r // 2, (r % 2) * nblk + b, 0), r))
        for r in range(R)
    ]
    ns_specs = [
        pl.BlockSpec((NB,),
                     functools.partial(lambda r, b: (r * nblk + b,), r))
        for r in range(R)
    ]
    ownblk = OWNSZ // NB
    c_specs = [
        pl.BlockSpec((NB,),
                     functools.partial(
                         lambda r, b: ((r // 2) * ownblk + (r % 2) * nblk + b,), r))
        for r in range(R)
    ]
    full = lambda *s: pl.BlockSpec(s, lambda b: tuple(0 for _ in s))
    return pl.pallas_call(
        _tc2_body,
        grid=(nblk,),
        in_specs=(agg_specs + ns_specs + c_specs
                  + [full(R, F, H), full(R, H), full(R, H, H), full(R, H),
                     full(H, 2), full(1, 2)]),
        out_specs=pl.BlockSpec((1, 2), lambda b: (0, 0)),
        out_shape=jax.ShapeDtypeStruct((1, 2), jnp.float32),
        scratch_shapes=[pltpu.VMEM((R, H), jnp.float32)],
    )(*([aggn] * R), *([nsv] * R), *([c_flat] * R),
      W1, b1, W2, b2, Wc, bc2)


def kernel(x, edge_index, edge_type, W1, b1, W2, b2, Wc, bc):
    src = edge_index[0].astype(jnp.int32)
    dst = edge_index[1].astype(jnp.int32)
    et = edge_type.astype(jnp.int32)
    e = src.shape[0]
    pad = EP - e
    spread = jnp.arange(pad, dtype=jnp.int32) & 2047
    srcp = jnp.concatenate([src, spread])
    dstp = jnp.concatenate([dst, spread])
    typp = jnp.concatenate([et, jnp.full((pad,), R, jnp.int32)])
    x_pad = jnp.pad(x.astype(jnp.float32), ((0, NP - N), (0, 0)))

    degs = _pass_a(srcp, dstp, typp)
    degs3 = degs.reshape(2, 2, DEGSZ)
    degs4 = degs.reshape(2, 2, DEGSZ // 16, 16)
    x_flat2d = x_pad.reshape(NP * F // 128, 128)
    nsv, ndv, xs2d = _tc1(degs3, degs4, x_flat2d)
    xs = xs2d.reshape(DEGSZ, F)
    agg_out, c_out = _pass_b(srcp, dstp, typp, xs, ndv)
    ndre2d = jnp.concatenate(
        [ndv[:OWNSZ], ndv[2 * NP:2 * NP + OWNSZ]]).reshape(2 * OWNSZ // 16, 16)
    aggn = _tc15(agg_out.reshape(2 * OWNSZ * F // 128, 128),
                 ndre2d).reshape(2, OWNSZ, F)
    out2 = _tc2(aggn, nsv, c_out.reshape(-1),
                W1.astype(jnp.float32), b1.astype(jnp.float32),
                W2.astype(jnp.float32), b2.astype(jnp.float32),
                Wc.astype(jnp.float32), bc.astype(jnp.float32).reshape(1, 2))
    return out2.reshape(2)
